# baseline (speedup 1.0000x reference)
import functools

import jax
import jax.numpy as jnp
from jax import lax
from jax.experimental import pallas as pl
from jax.experimental.pallas import tpu as pltpu
from jax.experimental.pallas import tpu_sc as plsc

_B = 16384
_D = 128
_NC = 2
_NS = 16
_NW = _NC * _NS
_BPW = _B // _NW
_CB = 128
_NCHUNK = _BPW // _CB
_NSLOT = 2
_AHEAD = 1
_GP = 16
_UNROLL = 4

_mesh = plsc.VectorSubcoreMesh(core_axis_name="c", subcore_axis_name="s",
                               num_cores=_NC)


def _compute_chunk(t_buf, r_buf, ch, out_v):
    last_lane = lax.iota(jnp.int32, _GP) == (_GP - 1)

    @plsc.parallel_loop(0, _CB, 1, unroll=_UNROLL)
    def row_body(r):
        prods = [t_buf[r, pl.ds(j * 16, 16)] * r_buf[r, pl.ds(j * 16, 16)]
                 for j in range(_D // 16)]
        while len(prods) > 1:
            prods = [prods[i] + prods[i + 1] for i in range(0, len(prods), 2)]
        total = plsc.cumsum(prods[0])
        plsc.store_compressed(out_v.at[pl.ds(ch * _CB + r, _GP)], total,
                              mask=last_lane)


@functools.partial(
    pl.kernel,
    mesh=_mesh,
    out_type=jax.ShapeDtypeStruct((_B,), jnp.float32),
    scratch_types=[
        pltpu.VMEM((_BPW,), jnp.int32),
        pltpu.VMEM((_BPW + _GP,), jnp.float32),
        pltpu.VMEM((_CB, _D), jnp.float32),
        pltpu.VMEM((_CB, _D), jnp.float32),
        pltpu.VMEM((_CB, _D), jnp.float32),
        pltpu.VMEM((_CB, _D), jnp.float32),
        pltpu.SemaphoreType.DMA,
        pltpu.SemaphoreType.DMA,
    ],
    compiler_params=pltpu.CompilerParams(needs_layout_passes=False),
)
def _spline_dot_sc(t_hbm, c_hbm, w_hbm, out_hbm,
                   idx_v, out_v, r0, r1, t0, t1, s0, s1):
    wid = lax.axis_index("s") * _NC + lax.axis_index("c")
    base = wid * _BPW

    rbufs = (r0, r1)
    tbufs = (t0, t1)
    sems = (s0, s1)

    def fire(ch):
        slot = ch % _NSLOT
        hr = pltpu.async_copy(
            w_hbm.at[idx_v.at[pl.ds(ch * _CB, _CB)]], rbufs[slot], sems[slot])
        ht = pltpu.async_copy(
            t_hbm.at[pl.ds(base + ch * _CB, _CB)], tbufs[slot], sems[slot])
        return hr, ht

    pltpu.sync_copy(c_hbm.at[pl.ds(base, _BPW)], idx_v)
    hr0 = pltpu.async_copy(w_hbm.at[idx_v.at[pl.ds(0, _CB)]], rbufs[0], sems[0])
    ht0 = pltpu.async_copy(t_hbm.at[pl.ds(base, _CB)], tbufs[0], sems[0])
    pending = [(hr0, ht0)] + [fire(ch) for ch in range(1, _AHEAD)]
    for ch in range(_NCHUNK):
        slot = ch % _NSLOT
        if ch + _AHEAD < _NCHUNK:
            pending.append(fire(ch + _AHEAD))
        hr, ht = pending.pop(0)
        hr.wait()
        ht.wait()
        _compute_chunk(tbufs[slot], rbufs[slot], ch, out_v)

    pltpu.sync_copy(out_v.at[pl.ds(0, _BPW)], out_hbm.at[pl.ds(base, _BPW)])


def kernel(t, c, alphas_weight):
    return _spline_dot_sc(t, c.astype(jnp.int32), alphas_weight)


# speedup vs baseline: 1.0168x; 1.0168x over previous
import functools

import jax
import jax.numpy as jnp
from jax import lax
from jax.experimental import pallas as pl
from jax.experimental.pallas import tpu as pltpu
from jax.experimental.pallas import tpu_sc as plsc

_B = 16384
_D = 128
_NC = 2
_NS = 16
_NW = _NC * _NS
_BPW = _B // _NW
_CB = 128
_NCHUNK = _BPW // _CB
_NSLOT = 2
_AHEAD = 1
_GP = 16
_UNROLL = 4

_mesh = plsc.VectorSubcoreMesh(core_axis_name="c", subcore_axis_name="s",
                               num_cores=_NC)


def _compute_chunk(t_buf, r_buf, ch, out_v):
    last_lane = lax.iota(jnp.int32, _GP) == (_GP - 1)

    @plsc.parallel_loop(0, _CB, 1, unroll=_UNROLL)
    def row_body(r):
        prods = [t_buf[r, pl.ds(j * 16, 16)] * r_buf[r, pl.ds(j * 16, 16)]
                 for j in range(_D // 16)]
        while len(prods) > 1:
            prods = [prods[i] + prods[i + 1] for i in range(0, len(prods), 2)]
        total = plsc.cumsum(prods[0])
        plsc.store_compressed(out_v.at[pl.ds(ch * _CB + r, _GP)], total,
                              mask=last_lane)


@functools.partial(
    pl.kernel,
    mesh=_mesh,
    out_type=jax.ShapeDtypeStruct((_B,), jnp.float32),
    scratch_types=[
        pltpu.VMEM((_BPW,), jnp.int32),
        pltpu.VMEM((_BPW + _GP,), jnp.float32),
        pltpu.VMEM((_CB, _D), jnp.float32),
        pltpu.VMEM((_CB, _D), jnp.float32),
        pltpu.VMEM((_CB, _D), jnp.float32),
        pltpu.VMEM((_CB, _D), jnp.float32),
        pltpu.SemaphoreType.DMA,
        pltpu.SemaphoreType.DMA,
    ],
    compiler_params=pltpu.CompilerParams(needs_layout_passes=False),
)
def _spline_dot_sc(t_hbm, c_hbm, w_hbm, out_hbm,
                   idx_v, out_v, r0, r1, t0, t1, s0, s1):
    wid = lax.axis_index("s") * _NC + lax.axis_index("c")
    base = wid * _BPW

    rbufs = (r0, r1)
    tbufs = (t0, t1)
    sems = (s0, s1)

    def fire(ch):
        slot = ch % _NSLOT
        hr = pltpu.async_copy(
            w_hbm.at[idx_v.at[pl.ds(ch * _CB, _CB)]], rbufs[slot], sems[slot])
        ht = pltpu.async_copy(
            t_hbm.at[pl.ds(base + ch * _CB, _CB)], tbufs[slot], sems[slot])
        return hr, ht

    ht0 = pltpu.async_copy(t_hbm.at[pl.ds(base, _CB)], tbufs[0], sems[0])
    pltpu.sync_copy(c_hbm.at[pl.ds(base, _BPW)], idx_v)
    hr0 = pltpu.async_copy(w_hbm.at[idx_v.at[pl.ds(0, _CB)]], rbufs[0], sems[0])
    pending = [(hr0, ht0)] + [fire(ch) for ch in range(1, _AHEAD)]
    for ch in range(_NCHUNK):
        slot = ch % _NSLOT
        if ch + _AHEAD < _NCHUNK:
            pending.append(fire(ch + _AHEAD))
        hr, ht = pending.pop(0)
        hr.wait()
        ht.wait()
        _compute_chunk(tbufs[slot], rbufs[slot], ch, out_v)

    pltpu.sync_copy(out_v.at[pl.ds(0, _BPW)], out_hbm.at[pl.ds(base, _BPW)])


def kernel(t, c, alphas_weight):
    return _spline_dot_sc(t, c.astype(jnp.int32), alphas_weight)


# speedup vs baseline: 1.0275x; 1.0105x over previous
import functools

import jax
import jax.numpy as jnp
from jax import lax
from jax.experimental import pallas as pl
from jax.experimental.pallas import tpu as pltpu
from jax.experimental.pallas import tpu_sc as plsc

_B = 16384
_D = 128
_NC = 2
_NS = 16
_NW = _NC * _NS
_BPW = _B // _NW
_CB = 128
_NCHUNK = _BPW // _CB
_NSLOT = 2
_AHEAD = 1
_GP = 16
_UNROLL = 2

_mesh = plsc.VectorSubcoreMesh(core_axis_name="c", subcore_axis_name="s",
                               num_cores=_NC)


def _compute_chunk(t_buf, r_buf, ch, out_v):
    last_lane = lax.iota(jnp.int32, _GP) == (_GP - 1)

    @plsc.parallel_loop(0, _CB, 1, unroll=_UNROLL)
    def row_body(r):
        prods = [t_buf[r, pl.ds(j * 16, 16)] * r_buf[r, pl.ds(j * 16, 16)]
                 for j in range(_D // 16)]
        while len(prods) > 1:
            prods = [prods[i] + prods[i + 1] for i in range(0, len(prods), 2)]
        total = plsc.cumsum(prods[0])
        plsc.store_compressed(out_v.at[pl.ds(ch * _CB + r, _GP)], total,
                              mask=last_lane)


@functools.partial(
    pl.kernel,
    mesh=_mesh,
    out_type=jax.ShapeDtypeStruct((_B,), jnp.float32),
    scratch_types=[
        pltpu.VMEM((_BPW,), jnp.int32),
        pltpu.VMEM((_BPW + _GP,), jnp.float32),
        pltpu.VMEM((_CB, _D), jnp.float32),
        pltpu.VMEM((_CB, _D), jnp.float32),
        pltpu.VMEM((_CB, _D), jnp.float32),
        pltpu.VMEM((_CB, _D), jnp.float32),
        pltpu.SemaphoreType.DMA,
        pltpu.SemaphoreType.DMA,
    ],
    compiler_params=pltpu.CompilerParams(needs_layout_passes=False),
)
def _spline_dot_sc(t_hbm, c_hbm, w_hbm, out_hbm,
                   idx_v, out_v, r0, r1, t0, t1, s0, s1):
    wid = lax.axis_index("s") * _NC + lax.axis_index("c")
    base = wid * _BPW

    rbufs = (r0, r1)
    tbufs = (t0, t1)
    sems = (s0, s1)

    def fire(ch):
        slot = ch % _NSLOT
        hr = pltpu.async_copy(
            w_hbm.at[idx_v.at[pl.ds(ch * _CB, _CB)]], rbufs[slot], sems[slot])
        ht = pltpu.async_copy(
            t_hbm.at[pl.ds(base + ch * _CB, _CB)], tbufs[slot], sems[slot])
        return hr, ht

    ht0 = pltpu.async_copy(t_hbm.at[pl.ds(base, _CB)], tbufs[0], sems[0])
    pltpu.sync_copy(c_hbm.at[pl.ds(base, _BPW)], idx_v)
    hr0 = pltpu.async_copy(w_hbm.at[idx_v.at[pl.ds(0, _CB)]], rbufs[0], sems[0])
    pending = [(hr0, ht0)] + [fire(ch) for ch in range(1, _AHEAD)]
    for ch in range(_NCHUNK):
        slot = ch % _NSLOT
        if ch + _AHEAD < _NCHUNK:
            pending.append(fire(ch + _AHEAD))
        hr, ht = pending.pop(0)
        hr.wait()
        ht.wait()
        _compute_chunk(tbufs[slot], rbufs[slot], ch, out_v)

    pltpu.sync_copy(out_v.at[pl.ds(0, _BPW)], out_hbm.at[pl.ds(base, _BPW)])


def kernel(t, c, alphas_weight):
    return _spline_dot_sc(t, c.astype(jnp.int32), alphas_weight)


# speedup vs baseline: 1.0357x; 1.0079x over previous
import functools

import jax
import jax.numpy as jnp
from jax import lax
from jax.experimental import pallas as pl
from jax.experimental.pallas import tpu as pltpu
from jax.experimental.pallas import tpu_sc as plsc

_B = 16384
_D = 128
_NC = 2
_NS = 16
_NW = _NC * _NS
_BPW = _B // _NW
_CB = 128
_NCHUNK = _BPW // _CB
_NSLOT = 2
_AHEAD = 1
_GP = 16
_UNROLL = 1

_mesh = plsc.VectorSubcoreMesh(core_axis_name="c", subcore_axis_name="s",
                               num_cores=_NC)


def _compute_chunk(t_buf, r_buf, ch, out_v):
    last_lane = lax.iota(jnp.int32, _GP) == (_GP - 1)

    @plsc.parallel_loop(0, _CB, 1, unroll=_UNROLL)
    def row_body(r):
        prods = [t_buf[r, pl.ds(j * 16, 16)] * r_buf[r, pl.ds(j * 16, 16)]
                 for j in range(_D // 16)]
        while len(prods) > 1:
            prods = [prods[i] + prods[i + 1] for i in range(0, len(prods), 2)]
        total = plsc.cumsum(prods[0])
        plsc.store_compressed(out_v.at[pl.ds(ch * _CB + r, _GP)], total,
                              mask=last_lane)


@functools.partial(
    pl.kernel,
    mesh=_mesh,
    out_type=jax.ShapeDtypeStruct((_B,), jnp.float32),
    scratch_types=[
        pltpu.VMEM((_BPW,), jnp.int32),
        pltpu.VMEM((_BPW + _GP,), jnp.float32),
        pltpu.VMEM((_CB, _D), jnp.float32),
        pltpu.VMEM((_CB, _D), jnp.float32),
        pltpu.VMEM((_CB, _D), jnp.float32),
        pltpu.VMEM((_CB, _D), jnp.float32),
        pltpu.SemaphoreType.DMA,
        pltpu.SemaphoreType.DMA,
    ],
    compiler_params=pltpu.CompilerParams(needs_layout_passes=False),
)
def _spline_dot_sc(t_hbm, c_hbm, w_hbm, out_hbm,
                   idx_v, out_v, r0, r1, t0, t1, s0, s1):
    wid = lax.axis_index("s") * _NC + lax.axis_index("c")
    base = wid * _BPW

    rbufs = (r0, r1)
    tbufs = (t0, t1)
    sems = (s0, s1)

    def fire(ch):
        slot = ch % _NSLOT
        hr = pltpu.async_copy(
            w_hbm.at[idx_v.at[pl.ds(ch * _CB, _CB)]], rbufs[slot], sems[slot])
        ht = pltpu.async_copy(
            t_hbm.at[pl.ds(base + ch * _CB, _CB)], tbufs[slot], sems[slot])
        return hr, ht

    ht0 = pltpu.async_copy(t_hbm.at[pl.ds(base, _CB)], tbufs[0], sems[0])
    pltpu.sync_copy(c_hbm.at[pl.ds(base, _BPW)], idx_v)
    hr0 = pltpu.async_copy(w_hbm.at[idx_v.at[pl.ds(0, _CB)]], rbufs[0], sems[0])
    pending = [(hr0, ht0)] + [fire(ch) for ch in range(1, _AHEAD)]
    for ch in range(_NCHUNK):
        slot = ch % _NSLOT
        if ch + _AHEAD < _NCHUNK:
            pending.append(fire(ch + _AHEAD))
        hr, ht = pending.pop(0)
        hr.wait()
        ht.wait()
        _compute_chunk(tbufs[slot], rbufs[slot], ch, out_v)

    pltpu.sync_copy(out_v.at[pl.ds(0, _BPW)], out_hbm.at[pl.ds(base, _BPW)])


def kernel(t, c, alphas_weight):
    return _spline_dot_sc(t, c.astype(jnp.int32), alphas_weight)
